# baseline (speedup 1.0000x reference)
import functools

import jax
import jax.numpy as jnp
from jax import lax
from jax.experimental import pallas as pl
from jax.experimental.pallas import tpu as pltpu
from jax.experimental.pallas import tpu_sc as plsc

B = 16384
HIST = 200
D = 64
NC, NS = 2, 16
NW = NC * NS
SPW = B // NW
GROUP = 64
NGROUPS = SPW // GROUP
C0, C1 = 128, 72


VOCAB = 1000000
CONV_COLS = 32768
CONV_GRID = -(-VOCAB // CONV_COLS)
VOCAB_PAD = CONV_GRID * CONV_COLS


def _conv_body(et_ref, id_ref, o_ref):
    t = lax.dot_general(et_ref[...], id_ref[...], (((0,), (0,)), ((), ())),
                        preferred_element_type=jnp.float32)
    h = CONV_COLS // 2
    o_ref[...] = jnp.concatenate([t[:h], t[h:]], axis=1)


NBUF = 6


def _pool_body(x_hbm, emb_hbm, out_hbm, idx_v, idx_t, rows0, rows1, rows2,
               rows3, rows4, rows5, out_v, sem0, sem1, sem2, sem3, sem4, sem5):
    wid = lax.axis_index("s") * NC + lax.axis_index("c")
    base = wid * SPW
    rows = (rows0, rows1, rows2, rows3, rows4, rows5)
    sems = (sem0, sem1, sem2, sem3, sem4, sem5)

    def issue(s, buf):
        a = pltpu.async_copy(
            emb_hbm.at[idx_t.at[s]], rows[buf], sems[buf])
        return (a,)

    def accum(buf, s):
        r = rows[buf]

        def body(j, accs):
            j2 = j * 2
            return tuple(accs[4 * rr + c] + r[j2 + rr, pl.ds(16 * c, 16)]
                         for rr in range(2) for c in range(4))

        z = jnp.zeros((16,), jnp.float32)
        a = lax.fori_loop(0, HIST // 2, body, (z,) * 8, unroll=4)
        for c in range(4):
            out_v[s // 2, pl.ds(64 * (s % 2) + 16 * c, 16)] = a[c] + a[4 + c]

    XCOLS = [16 * t for t in range(HIST // 16)] + [HIST - 16]

    def xform(s, carry):
        for c in XCOLS:
            v = idx_v[s, pl.ds(c, 16)]
            lo = v & (CONV_COLS - 1)
            idx_t[s, pl.ds(c, 16)] = (
                v - lo + ((lo & (CONV_COLS // 2 - 1)) << 1)
                + (lo >> (CONV_COLS.bit_length() - 2)))
        return carry

    def group(g, carry):
        row0 = base + g * GROUP
        pltpu.sync_copy(x_hbm.at[pl.ds(row0, GROUP)], idx_v)
        lax.fori_loop(0, GROUP, xform, 0)
        pending = [issue(s, s % NBUF) for s in range(NBUF - 1)]
        for s in range(GROUP):
            for d in pending[0]:
                d.wait()
            pending = pending[1:]
            if s + NBUF - 1 < GROUP:
                pending.append(issue(s + NBUF - 1, (s + NBUF - 1) % NBUF))
            accum(s % NBUF, s)
        pltpu.sync_copy(out_v, out_hbm.at[pl.ds(row0 // 2, GROUP // 2)])
        return carry

    lax.fori_loop(0, NGROUPS, group, 0)


_conv = pl.pallas_call(
    _conv_body,
    grid=(CONV_GRID,),
    in_specs=[
        pl.BlockSpec((D, CONV_COLS), lambda i: (0, i)),
        pl.BlockSpec((D, D), lambda i: (0, 0)),
    ],
    out_specs=pl.BlockSpec((CONV_COLS // 2, 128), lambda i: (i, 0)),
    out_shape=jax.ShapeDtypeStruct((VOCAB_PAD // 2, 128), jnp.float32),
)


_pool = pl.kernel(
    _pool_body,
    out_type=jax.ShapeDtypeStruct((B // 2, 2 * D), jnp.float32),
    mesh=plsc.VectorSubcoreMesh(core_axis_name="c", subcore_axis_name="s"),
    scratch_types=[
        pltpu.VMEM((GROUP, HIST), jnp.int32),
        pltpu.VMEM((GROUP, HIST), jnp.int32),
        pltpu.VMEM((HIST, D), jnp.float32),
        pltpu.VMEM((HIST, D), jnp.float32),
        pltpu.VMEM((HIST, D), jnp.float32),
        pltpu.VMEM((HIST, D), jnp.float32),
        pltpu.VMEM((HIST, D), jnp.float32),
        pltpu.VMEM((HIST, D), jnp.float32),
        pltpu.VMEM((GROUP // 2, 2 * D), jnp.float32),
        pltpu.SemaphoreType.DMA,
        pltpu.SemaphoreType.DMA,
        pltpu.SemaphoreType.DMA,
        pltpu.SemaphoreType.DMA,
        pltpu.SemaphoreType.DMA,
        pltpu.SemaphoreType.DMA,
    ],
    compiler_params=pltpu.CompilerParams(use_tc_tiling_on_sc=False),
)


def _mlp_body(s_ref, w1_ref, b1_ref, w2_ref, b2_ref, o_ref):
    h = jnp.dot(s_ref[...], w1_ref[...], preferred_element_type=jnp.float32)
    h = jnp.maximum(h + b1_ref[...], 0.0)
    z = h * w2_ref[...]
    o_ref[...] = jnp.concatenate(
        [jnp.sum(z[:, :D], axis=1, keepdims=True),
         jnp.sum(z[:, D:], axis=1, keepdims=True)], axis=1) + b2_ref[...]


MLP_BLK = 2048

_mlp = pl.pallas_call(
    _mlp_body,
    grid=(B // 2 // MLP_BLK,),
    in_specs=[
        pl.BlockSpec((MLP_BLK, 2 * D), lambda i: (i, 0)),
        pl.BlockSpec((2 * D, 2 * D), lambda i: (0, 0)),
        pl.BlockSpec((1, 2 * D), lambda i: (0, 0)),
        pl.BlockSpec((1, 2 * D), lambda i: (0, 0)),
        pl.BlockSpec((1, 2), lambda i: (0, 0)),
    ],
    out_specs=pl.BlockSpec((MLP_BLK, 2), lambda i: (i, 0)),
    out_shape=jax.ShapeDtypeStruct((B // 2, 2), jnp.float32),
)


def kernel(x, embeddings, W1, b1, W2, b2):
    emb_lin = _conv(embeddings.T, jnp.eye(D, dtype=jnp.float32))
    emb_lin = emb_lin.reshape(VOCAB_PAD, D)
    sums2 = _pool(x.astype(jnp.int32), emb_lin)
    w1s = W1.astype(jnp.float32) * (1.0 / HIST)
    zero = jnp.zeros((D, D), jnp.float32)
    w1d = jnp.block([[w1s, zero], [zero, w1s]])
    b1d = jnp.tile(b1.reshape(1, D), (1, 2))
    w2d = jnp.tile(W2.reshape(1, D), (1, 2))
    b2d = jnp.tile(b2.reshape(1, 1), (1, 2))
    out2 = _mlp(sums2, w1d, b1d, w2d, b2d)
    return out2.reshape(B, 1)


# speedup vs baseline: 1.0009x; 1.0009x over previous
import functools

import jax
import jax.numpy as jnp
from jax import lax
from jax.experimental import pallas as pl
from jax.experimental.pallas import tpu as pltpu
from jax.experimental.pallas import tpu_sc as plsc

B = 16384
HIST = 200
D = 64
NC, NS = 2, 16
NW = NC * NS
SPW = B // NW
GROUP = 64
NGROUPS = SPW // GROUP
C0, C1 = 128, 72


VOCAB = 1000000
CONV_COLS = 32768
CONV_GRID = -(-VOCAB // CONV_COLS)
VOCAB_PAD = CONV_GRID * CONV_COLS


def _conv_body(et_ref, id_ref, o_ref):
    t = lax.dot_general(et_ref[...], id_ref[...], (((0,), (0,)), ((), ())),
                        preferred_element_type=jnp.float32)
    h = CONV_COLS // 2
    o_ref[...] = jnp.concatenate([t[:h], t[h:]], axis=1)


NBUF = 6


def _pool_body(x_hbm, emb_hbm, out_hbm, idx_v, idx_t, rows0, rows1, rows2,
               rows3, rows4, rows5, out_v, sem0, sem1, sem2, sem3, sem4, sem5):
    wid = lax.axis_index("s") * NC + lax.axis_index("c")
    base = wid * SPW
    rows = (rows0, rows1, rows2, rows3, rows4, rows5)
    sems = (sem0, sem1, sem2, sem3, sem4, sem5)

    def issue(s, buf):
        a = pltpu.async_copy(
            emb_hbm.at[idx_t.at[s, pl.ds(0, C0)]],
            rows[buf].at[pl.ds(0, C0)], sems[buf])
        b = pltpu.async_copy(
            emb_hbm.at[idx_t.at[s, pl.ds(C0, C1)]],
            rows[buf].at[pl.ds(C0, C1)], sems[buf])
        return a, b

    def accum(buf, s):
        r = rows[buf]

        def body(j, accs):
            j2 = j * 2
            return tuple(accs[4 * rr + c] + r[j2 + rr, pl.ds(16 * c, 16)]
                         for rr in range(2) for c in range(4))

        z = jnp.zeros((16,), jnp.float32)
        a = lax.fori_loop(0, HIST // 2, body, (z,) * 8, unroll=4)
        for c in range(4):
            out_v[s // 2, pl.ds(64 * (s % 2) + 16 * c, 16)] = a[c] + a[4 + c]

    XCOLS = [16 * t for t in range(HIST // 16)] + [HIST - 16]

    def xform(s, carry):
        for c in XCOLS:
            v = idx_v[s, pl.ds(c, 16)]
            lo = v & (CONV_COLS - 1)
            idx_t[s, pl.ds(c, 16)] = (
                v - lo + ((lo & (CONV_COLS // 2 - 1)) << 1)
                + (lo >> (CONV_COLS.bit_length() - 2)))
        return carry

    def group(g, carry):
        row0 = base + g * GROUP
        pltpu.sync_copy(x_hbm.at[pl.ds(row0, GROUP)], idx_v)
        lax.fori_loop(0, GROUP, xform, 0)
        pending = [issue(s, s % NBUF) for s in range(NBUF - 1)]
        for s in range(GROUP):
            for d in pending[0]:
                d.wait()
            pending = pending[1:]
            if s + NBUF - 1 < GROUP:
                pending.append(issue(s + NBUF - 1, (s + NBUF - 1) % NBUF))
            accum(s % NBUF, s)
        pltpu.sync_copy(out_v, out_hbm.at[pl.ds(row0 // 2, GROUP // 2)])
        return carry

    lax.fori_loop(0, NGROUPS, group, 0)


_conv = pl.pallas_call(
    _conv_body,
    grid=(CONV_GRID,),
    in_specs=[
        pl.BlockSpec((D, CONV_COLS), lambda i: (0, i)),
        pl.BlockSpec((D, D), lambda i: (0, 0)),
    ],
    out_specs=pl.BlockSpec((CONV_COLS // 2, 128), lambda i: (i, 0)),
    out_shape=jax.ShapeDtypeStruct((VOCAB_PAD // 2, 128), jnp.float32),
)


_pool = pl.kernel(
    _pool_body,
    out_type=jax.ShapeDtypeStruct((B // 2, 2 * D), jnp.float32),
    mesh=plsc.VectorSubcoreMesh(core_axis_name="c", subcore_axis_name="s"),
    scratch_types=[
        pltpu.VMEM((GROUP, HIST), jnp.int32),
        pltpu.VMEM((GROUP, HIST), jnp.int32),
        pltpu.VMEM((HIST, D), jnp.float32),
        pltpu.VMEM((HIST, D), jnp.float32),
        pltpu.VMEM((HIST, D), jnp.float32),
        pltpu.VMEM((HIST, D), jnp.float32),
        pltpu.VMEM((HIST, D), jnp.float32),
        pltpu.VMEM((HIST, D), jnp.float32),
        pltpu.VMEM((GROUP // 2, 2 * D), jnp.float32),
        pltpu.SemaphoreType.DMA,
        pltpu.SemaphoreType.DMA,
        pltpu.SemaphoreType.DMA,
        pltpu.SemaphoreType.DMA,
        pltpu.SemaphoreType.DMA,
        pltpu.SemaphoreType.DMA,
    ],
    compiler_params=pltpu.CompilerParams(use_tc_tiling_on_sc=False),
)


def _mlp_body(s_ref, w1_ref, b1_ref, w2_ref, b2_ref, o_ref):
    h = jnp.dot(s_ref[...], w1_ref[...], preferred_element_type=jnp.float32)
    h = jnp.maximum(h + b1_ref[...], 0.0)
    z = h * w2_ref[...]
    o_ref[...] = jnp.concatenate(
        [jnp.sum(z[:, :D], axis=1, keepdims=True),
         jnp.sum(z[:, D:], axis=1, keepdims=True)], axis=1) + b2_ref[...]


MLP_BLK = 2048

_mlp = pl.pallas_call(
    _mlp_body,
    grid=(B // 2 // MLP_BLK,),
    in_specs=[
        pl.BlockSpec((MLP_BLK, 2 * D), lambda i: (i, 0)),
        pl.BlockSpec((2 * D, 2 * D), lambda i: (0, 0)),
        pl.BlockSpec((1, 2 * D), lambda i: (0, 0)),
        pl.BlockSpec((1, 2 * D), lambda i: (0, 0)),
        pl.BlockSpec((1, 2), lambda i: (0, 0)),
    ],
    out_specs=pl.BlockSpec((MLP_BLK, 2), lambda i: (i, 0)),
    out_shape=jax.ShapeDtypeStruct((B // 2, 2), jnp.float32),
)


def kernel(x, embeddings, W1, b1, W2, b2):
    emb_lin = _conv(embeddings.T, jnp.eye(D, dtype=jnp.float32))
    emb_lin = emb_lin.reshape(VOCAB_PAD, D)
    sums2 = _pool(x.astype(jnp.int32), emb_lin)
    w1s = W1.astype(jnp.float32) * (1.0 / HIST)
    zero = jnp.zeros((D, D), jnp.float32)
    w1d = jnp.block([[w1s, zero], [zero, w1s]])
    b1d = jnp.tile(b1.reshape(1, D), (1, 2))
    w2d = jnp.tile(W2.reshape(1, D), (1, 2))
    b2d = jnp.tile(b2.reshape(1, 1), (1, 2))
    out2 = _mlp(sums2, w1d, b1d, w2d, b2d)
    return out2.reshape(B, 1)


# speedup vs baseline: 1.0018x; 1.0009x over previous
import jax
import jax.numpy as jnp
from jax import lax
from jax.experimental import pallas as pl
from jax.experimental.pallas import tpu as pltpu
from jax.experimental.pallas import tpu_sc as plsc

B = 16384
HIST = 200
D = 64
NC, NS = 2, 16
NW = NC * NS
SPW = B // NW
GROUP = 64
NGROUPS = SPW // GROUP
C0, C1 = 128, 72


VOCAB = 1000000
CONV_COLS = 32768
CONV_GRID = -(-VOCAB // CONV_COLS)
VOCAB_PAD = CONV_GRID * CONV_COLS


def _conv_body(et_ref, id_ref, o_ref):
    t = lax.dot_general(et_ref[...], id_ref[...], (((0,), (0,)), ((), ())),
                        preferred_element_type=jnp.float32)
    h = CONV_COLS // 2
    o_ref[...] = jnp.concatenate([t[:h], t[h:]], axis=1)


NBUF = 6


def _pool_body(x_hbm, emb_hbm, out_hbm, idx_v, idx_t, rows0, rows1, rows2,
               rows3, rows4, rows5, out_v, sem0, sem1, sem2, sem3, sem4, sem5):
    wid = lax.axis_index("s") * NC + lax.axis_index("c")
    base = wid * SPW
    rows = (rows0, rows1, rows2, rows3, rows4, rows5)
    sems = (sem0, sem1, sem2, sem3, sem4, sem5)

    def issue(s, buf):
        a = pltpu.async_copy(
            emb_hbm.at[idx_t.at[s, pl.ds(0, C0)]],
            rows[buf].at[pl.ds(0, C0)], sems[buf])
        b = pltpu.async_copy(
            emb_hbm.at[idx_t.at[s, pl.ds(C0, C1)]],
            rows[buf].at[pl.ds(C0, C1)], sems[buf])
        return a, b

    def accum(buf, s):
        r = rows[buf]

        def body(j, accs):
            j2 = j * 2
            return tuple(accs[4 * rr + c] + r[j2 + rr, pl.ds(16 * c, 16)]
                         for rr in range(2) for c in range(4))

        z = jnp.zeros((16,), jnp.float32)
        a = lax.fori_loop(0, HIST // 2, body, (z,) * 8, unroll=4)
        for c in range(4):
            out_v[s // 2, pl.ds(64 * (s % 2) + 16 * c, 16)] = a[c] + a[4 + c]

    XCOLS = [16 * t for t in range(HIST // 16)] + [HIST - 16]

    def xform(s, carry):
        for c in XCOLS:
            v = idx_v[s, pl.ds(c, 16)]
            lo = v & (CONV_COLS - 1)
            idx_t[s, pl.ds(c, 16)] = (
                v - lo + ((lo & (CONV_COLS // 2 - 1)) << 1)
                + (lo >> (CONV_COLS.bit_length() - 2)))
        return carry

    def group(g, carry):
        row0 = base + g * GROUP
        pltpu.sync_copy(x_hbm.at[pl.ds(row0, GROUP)], idx_v)
        lax.fori_loop(0, GROUP, xform, 0)
        pending = [issue(s, s % NBUF) for s in range(NBUF - 1)]
        for s in range(GROUP):
            for d in pending[0]:
                d.wait()
            pending = pending[1:]
            if s + NBUF - 1 < GROUP:
                pending.append(issue(s + NBUF - 1, (s + NBUF - 1) % NBUF))
            accum(s % NBUF, s)
        pltpu.sync_copy(out_v, out_hbm.at[pl.ds(row0 // 2, GROUP // 2)])
        return carry

    lax.fori_loop(0, NGROUPS, group, 0)


_conv = pl.pallas_call(
    _conv_body,
    grid=(CONV_GRID,),
    in_specs=[
        pl.BlockSpec((D, CONV_COLS), lambda i: (0, i)),
        pl.BlockSpec((D, D), lambda i: (0, 0)),
    ],
    out_specs=pl.BlockSpec((CONV_COLS // 2, 128), lambda i: (i, 0)),
    out_shape=jax.ShapeDtypeStruct((VOCAB_PAD // 2, 128), jnp.float32),
)


_pool = pl.kernel(
    _pool_body,
    out_type=jax.ShapeDtypeStruct((B // 2, 2 * D), jnp.float32),
    mesh=plsc.VectorSubcoreMesh(core_axis_name="c", subcore_axis_name="s"),
    scratch_types=[
        pltpu.VMEM((GROUP, HIST), jnp.int32),
        pltpu.VMEM((GROUP, HIST), jnp.int32),
        pltpu.VMEM((HIST, D), jnp.float32),
        pltpu.VMEM((HIST, D), jnp.float32),
        pltpu.VMEM((HIST, D), jnp.float32),
        pltpu.VMEM((HIST, D), jnp.float32),
        pltpu.VMEM((HIST, D), jnp.float32),
        pltpu.VMEM((HIST, D), jnp.float32),
        pltpu.VMEM((GROUP // 2, 2 * D), jnp.float32),
        pltpu.SemaphoreType.DMA,
        pltpu.SemaphoreType.DMA,
        pltpu.SemaphoreType.DMA,
        pltpu.SemaphoreType.DMA,
        pltpu.SemaphoreType.DMA,
        pltpu.SemaphoreType.DMA,
    ],
    compiler_params=pltpu.CompilerParams(use_tc_tiling_on_sc=False),
)


def _mlp_body(s_ref, w1_ref, b1_ref, w2_ref, b2_ref, o_ref):
    h = jnp.dot(s_ref[...], w1_ref[...], preferred_element_type=jnp.float32)
    h = jnp.maximum(h + b1_ref[...], 0.0)
    z = h * w2_ref[...]
    o_ref[...] = jnp.concatenate(
        [jnp.sum(z[:, :D], axis=1, keepdims=True),
         jnp.sum(z[:, D:], axis=1, keepdims=True)], axis=1) + b2_ref[...]


MLP_BLK = 2048

_mlp = pl.pallas_call(
    _mlp_body,
    grid=(B // 2 // MLP_BLK,),
    in_specs=[
        pl.BlockSpec((MLP_BLK, 2 * D), lambda i: (i, 0)),
        pl.BlockSpec((2 * D, 2 * D), lambda i: (0, 0)),
        pl.BlockSpec((1, 2 * D), lambda i: (0, 0)),
        pl.BlockSpec((1, 2 * D), lambda i: (0, 0)),
        pl.BlockSpec((1, 2), lambda i: (0, 0)),
    ],
    out_specs=pl.BlockSpec((MLP_BLK, 2), lambda i: (i, 0)),
    out_shape=jax.ShapeDtypeStruct((B // 2, 2), jnp.float32),
)


def kernel(x, embeddings, W1, b1, W2, b2):
    emb_lin = _conv(embeddings.T, jnp.eye(D, dtype=jnp.float32))
    emb_lin = emb_lin.reshape(VOCAB_PAD, D)
    sums2 = _pool(x.astype(jnp.int32), emb_lin)
    w1s = W1.astype(jnp.float32) * (1.0 / HIST)
    zero = jnp.zeros((D, D), jnp.float32)
    w1d = jnp.block([[w1s, zero], [zero, w1s]])
    b1d = jnp.tile(b1.reshape(1, D), (1, 2))
    w2d = jnp.tile(W2.reshape(1, D), (1, 2))
    b2d = jnp.tile(b2.reshape(1, 1), (1, 2))
    out2 = _mlp(sums2, w1d, b1d, w2d, b2d)
    return out2.reshape(B, 1)
